# baseline (speedup 1.0000x reference)
import functools

import jax
import jax.numpy as jnp
from jax import lax
from jax.experimental import pallas as pl
from jax.experimental.pallas import tpu as pltpu
from jax.experimental.pallas import tpu_sc as plsc

_CH = 128
_SPLIT = 2


def _vq_tc_body(scale, d, zt_ref, wt_ref, idx_ref, loss_ref):
    pid = pl.program_id(0)
    zt = zt_ref[...].reshape(d, zt_ref.shape[-1])
    wt = wt_ref[...]
    s = zt.shape[1]
    k = wt.shape[1]
    zsq = jnp.sum(zt * zt, axis=0)
    wsq = jnp.sum(wt * wt, axis=0)
    wsq_col = wsq.reshape(k, 1)
    mm = lax.dot_general(wt, zt, (((0,), (0,)), ((), ())),
                         preferred_element_type=jnp.float32)
    dist = (zsq[None, :] + wsq_col) - 2.0 * mm
    val = dist[0:_CH, :]
    chk = jnp.zeros((_CH, s), jnp.float32)
    for j in range(1, k // _CH):
        dj = dist[j * _CH:(j + 1) * _CH, :]
        cond = dj < val
        val = jnp.minimum(val, dj)
        chk = jnp.where(cond, jnp.float32(j), chk)
    m = jnp.min(val, axis=0)
    row_f = lax.broadcasted_iota(jnp.int32, (_CH, s), 0).astype(jnp.float32)
    g = chk * jnp.float32(_CH) + row_f
    idxf = jnp.min(jnp.where(val == m[None, :], g, jnp.float32(k)), axis=0)
    idx_ref[...] = idxf.astype(jnp.int32)

    @pl.when(pid == 0)
    def _init():
        loss_ref[...] = jnp.zeros((1, 1), jnp.float32)

    loss_ref[...] += jnp.sum(m).reshape(1, 1)

    @pl.when(pid == pl.num_programs(0) - 1)
    def _finish():
        loss_ref[...] *= scale


def _tc_half(zt, wt, off, nb, scale):
    d, k = wt.shape
    s = zt.shape[-1]
    return pl.pallas_call(
        functools.partial(_vq_tc_body, scale, d),
        grid=(nb,),
        in_specs=[
            pl.BlockSpec((1, d, s), lambda i, off=off: (i + off, 0, 0)),
            pl.BlockSpec((d, k), lambda i: (0, 0)),
        ],
        out_specs=[
            pl.BlockSpec((s,), lambda i: (i,)),
            pl.BlockSpec((1, 1), lambda i: (0, 0)),
        ],
        out_shape=[
            jax.ShapeDtypeStruct((nb * s,), jnp.int32),
            jax.ShapeDtypeStruct((1, 1), jnp.float32),
        ],
    )(zt, wt)


def _sc_gather_t(wt, idx, nb, s):
    d, k = wt.shape
    hd = d // 2
    nt = nb * s
    info = plsc.get_sparse_core_info()
    nl = info.num_lanes
    nw = info.num_cores * info.num_subcores
    tpw = nt // nw
    mesh = plsc.VectorSubcoreMesh(core_axis_name="c", subcore_axis_name="s")

    @functools.partial(
        pl.kernel, mesh=mesh,
        out_type=jax.ShapeDtypeStruct((nb, d, s), jnp.float32),
        compiler_params=pltpu.CompilerParams(use_tc_tiling_on_sc=True,
                                             needs_layout_passes=False),
        scratch_types=[
            pltpu.VMEM((tpw,), jnp.int32),
            pltpu.VMEM((d, k), jnp.float32),
            pltpu.VMEM((hd, tpw), jnp.float32),
        ],
    )
    def gk(wt_hbm, idx_hbm, out_hbm, idx_v, wt_v, out_v):
        wid = lax.axis_index("s") * info.num_cores + lax.axis_index("c")
        bb = wid // (s // tpw)
        off = (wid % (s // tpw)) * tpw
        pltpu.sync_copy(wt_hbm, wt_v)
        pltpu.sync_copy(idx_hbm.at[pl.ds(wid * tpw, tpw)], idx_v)
        for h in range(2):
            for dd in range(hd):
                row = jnp.full((nl,), h * hd + dd, jnp.int32)

                @plsc.parallel_loop(0, tpw // nl, unroll=8)
                def _gather_row(gg, dd=dd, row=row):
                    toks = idx_v[pl.ds(gg * nl, nl)]
                    vals = plsc.load_gather(wt_v, [row, toks])
                    out_v[dd, pl.ds(gg * nl, nl)] = vals

            pltpu.sync_copy(out_v,
                            out_hbm.at[bb, pl.ds(h * hd, hd), pl.ds(off, tpw)])

    return gk(wt, idx)


def kernel(z, W):
    b, s, d = z.shape
    k = W.shape[0]
    zt = jnp.transpose(z, (0, 2, 1))
    wt = jnp.transpose(W, (1, 0))

    nh = b // _SPLIT
    scale = 1.25 / (b * s * d)
    idx_parts, loss_parts, qt_parts = [], [], []
    for p in range(_SPLIT):
        idx_p, lacc_p = _tc_half(zt, wt, p * nh, nh, scale)
        qt_parts.append(_sc_gather_t(wt, idx_p, nh, s))
        idx_parts.append(idx_p)
        loss_parts.append(lacc_p[0, 0])

    qt = jnp.concatenate(qt_parts, axis=0)
    idx = jnp.concatenate(idx_parts).reshape(b, s)
    loss = loss_parts[0]
    for lp in loss_parts[1:]:
        loss = loss + lp
    return jnp.transpose(qt, (0, 2, 1)), loss, idx


# speedup vs baseline: 1.4065x; 1.4065x over previous
import functools

import jax
import jax.numpy as jnp
from jax import lax
from jax.experimental import pallas as pl
from jax.experimental.pallas import tpu as pltpu
from jax.experimental.pallas import tpu_sc as plsc

_CH = 128
_SPLIT = 1


def _vq_tc_body(scale, d, zt_ref, wt_ref, idx_ref, loss_ref):
    pid = pl.program_id(0)
    zt = zt_ref[...].reshape(d, zt_ref.shape[-1])
    wt = wt_ref[...]
    s = zt.shape[1]
    k = wt.shape[1]
    zsq = jnp.sum(zt * zt, axis=0)
    wsq = jnp.sum(wt * wt, axis=0)
    wsq_col = wsq.reshape(k, 1)
    mm = lax.dot_general(wt, zt, (((0,), (0,)), ((), ())),
                         preferred_element_type=jnp.float32)
    dist = (zsq[None, :] + wsq_col) - 2.0 * mm
    val = dist[0:_CH, :]
    chk = jnp.zeros((_CH, s), jnp.float32)
    for j in range(1, k // _CH):
        dj = dist[j * _CH:(j + 1) * _CH, :]
        cond = dj < val
        val = jnp.minimum(val, dj)
        chk = jnp.where(cond, jnp.float32(j), chk)
    m = jnp.min(val, axis=0)
    row_f = lax.broadcasted_iota(jnp.int32, (_CH, s), 0).astype(jnp.float32)
    g = chk * jnp.float32(_CH) + row_f
    idxf = jnp.min(jnp.where(val == m[None, :], g, jnp.float32(k)), axis=0)
    idx_ref[...] = idxf.astype(jnp.int32)

    @pl.when(pid == 0)
    def _init():
        loss_ref[...] = jnp.zeros((1, 1), jnp.float32)

    loss_ref[...] += jnp.sum(m).reshape(1, 1)

    @pl.when(pid == pl.num_programs(0) - 1)
    def _finish():
        loss_ref[...] *= scale


def _tc_half(zt, wt, off, nb, scale):
    d, k = wt.shape
    s = zt.shape[-1]
    return pl.pallas_call(
        functools.partial(_vq_tc_body, scale, d),
        grid=(nb,),
        in_specs=[
            pl.BlockSpec((1, d, s), lambda i, off=off: (i + off, 0, 0)),
            pl.BlockSpec((d, k), lambda i: (0, 0)),
        ],
        out_specs=[
            pl.BlockSpec((s,), lambda i: (i,)),
            pl.BlockSpec((1, 1), lambda i: (0, 0)),
        ],
        out_shape=[
            jax.ShapeDtypeStruct((nb * s,), jnp.int32),
            jax.ShapeDtypeStruct((1, 1), jnp.float32),
        ],
    )(zt, wt)


def _sc_gather_t(wt, idx, nb, s):
    d, k = wt.shape
    hd = d // 2
    nt = nb * s
    info = plsc.get_sparse_core_info()
    nl = info.num_lanes
    nw = info.num_cores * info.num_subcores
    nrng = nw // 2
    tpw = nt // nrng
    mesh = plsc.VectorSubcoreMesh(core_axis_name="c", subcore_axis_name="s")

    @functools.partial(
        pl.kernel, mesh=mesh,
        out_type=jax.ShapeDtypeStruct((nb, d, s), jnp.float32),
        compiler_params=pltpu.CompilerParams(use_tc_tiling_on_sc=True,
                                             needs_layout_passes=False),
        scratch_types=[
            pltpu.VMEM((tpw,), jnp.int32),
            pltpu.VMEM((hd, k), jnp.float32),
            pltpu.VMEM((hd, tpw), jnp.float32),
        ],
    )
    def gk(wt_hbm, idx_hbm, out_hbm, idx_v, wt_v, out_v):
        wid = lax.axis_index("s") * info.num_cores + lax.axis_index("c")
        h = wid % 2
        rng = wid // 2
        tpb = tpw // s
        bb0 = rng * tpb
        pltpu.sync_copy(wt_hbm.at[pl.ds(h * hd, hd)], wt_v)
        pltpu.sync_copy(idx_hbm.at[pl.ds(rng * tpw, tpw)], idx_v)
        rows = [jnp.full((nl,), dd, jnp.int32) for dd in range(hd)]

        @plsc.parallel_loop(0, tpw // nl, unroll=2)
        def _gather_group(gg):
            toks = idx_v[pl.ds(gg * nl, nl)]
            for dd in range(hd):
                out_v[dd, pl.ds(gg * nl, nl)] = plsc.load_gather(
                    wt_v, [rows[dd], toks])

        for b2 in range(tpb):
            pltpu.sync_copy(out_v.at[pl.ds(0, hd), pl.ds(b2 * s, s)],
                            out_hbm.at[bb0 + b2, pl.ds(h * hd, hd)])

    return gk(wt, idx)


def kernel(z, W):
    b, s, d = z.shape
    k = W.shape[0]
    zt = jnp.transpose(z, (0, 2, 1))
    wt = jnp.transpose(W, (1, 0))

    nh = b // _SPLIT
    scale = 1.25 / (b * s * d)
    idx_parts, loss_parts, qt_parts = [], [], []
    for p in range(_SPLIT):
        idx_p, lacc_p = _tc_half(zt, wt, p * nh, nh, scale)
        qt_parts.append(_sc_gather_t(wt, idx_p, nh, s))
        idx_parts.append(idx_p)
        loss_parts.append(lacc_p[0, 0])

    qt = jnp.concatenate(qt_parts, axis=0)
    idx = jnp.concatenate(idx_parts).reshape(b, s)
    loss = loss_parts[0]
    for lp in loss_parts[1:]:
        loss = loss + lp
    return jnp.transpose(qt, (0, 2, 1)), loss, idx


# speedup vs baseline: 1.5002x; 1.0666x over previous
import functools

import jax
import jax.numpy as jnp
from jax import lax
from jax.experimental import pallas as pl
from jax.experimental.pallas import tpu as pltpu
from jax.experimental.pallas import tpu_sc as plsc

_CH = 128
_SPLIT = 1


def _vq_tc_body(scale, d, rb, zt_ref, wt_ref, idx_ref, loss_ref):
    pid = pl.program_id(0)
    wt = wt_ref[...]
    k = wt.shape[1]
    s = zt_ref.shape[-1]
    wsq = jnp.sum(wt * wt, axis=0)
    wsq_col = wsq.reshape(k, 1)
    row_f = lax.broadcasted_iota(jnp.int32, (_CH, s), 0).astype(jnp.float32)
    msum = jnp.zeros((), jnp.float32)
    for bi in range(rb):
        zt = zt_ref[bi]
        zsq = jnp.sum(zt * zt, axis=0)
        mm = lax.dot_general(wt, zt, (((0,), (0,)), ((), ())),
                             preferred_element_type=jnp.float32)
        dist = (zsq[None, :] + wsq_col) - 2.0 * mm
        val = dist[0:_CH, :]
        chk = jnp.zeros((_CH, s), jnp.float32)
        for j in range(1, k // _CH):
            dj = dist[j * _CH:(j + 1) * _CH, :]
            cond = dj < val
            val = jnp.minimum(val, dj)
            chk = jnp.where(cond, jnp.float32(j), chk)
        m = jnp.min(val, axis=0)
        g = chk * jnp.float32(_CH) + row_f
        idxf = jnp.min(jnp.where(val == m[None, :], g, jnp.float32(k)),
                       axis=0)
        idx_ref[pl.ds(bi * s, s)] = idxf.astype(jnp.int32)
        msum = msum + jnp.sum(m)

    @pl.when(pid == 0)
    def _init():
        loss_ref[...] = jnp.zeros((1, 1), jnp.float32)

    loss_ref[...] += msum.reshape(1, 1)

    @pl.when(pid == pl.num_programs(0) - 1)
    def _finish():
        loss_ref[...] *= scale


def _tc_half(zt, wt, off, nb, scale, rb=2):
    d, k = wt.shape
    s = zt.shape[-1]
    return pl.pallas_call(
        functools.partial(_vq_tc_body, scale, d, rb),
        grid=(nb // rb,),
        in_specs=[
            pl.BlockSpec((rb, d, s), lambda i, off=off: (i + off, 0, 0)),
            pl.BlockSpec((d, k), lambda i: (0, 0)),
        ],
        out_specs=[
            pl.BlockSpec((rb * s,), lambda i: (i,)),
            pl.BlockSpec((1, 1), lambda i: (0, 0)),
        ],
        out_shape=[
            jax.ShapeDtypeStruct((nb * s,), jnp.int32),
            jax.ShapeDtypeStruct((1, 1), jnp.float32),
        ],
    )(zt, wt)


def _sc_gather_t(wt, idx, nb, s):
    d, k = wt.shape
    hd = d // 2
    nt = nb * s
    info = plsc.get_sparse_core_info()
    nl = info.num_lanes
    nw = info.num_cores * info.num_subcores
    nrng = nw // 2
    tpw = nt // nrng
    mesh = plsc.VectorSubcoreMesh(core_axis_name="c", subcore_axis_name="s")

    @functools.partial(
        pl.kernel, mesh=mesh,
        out_type=jax.ShapeDtypeStruct((nb, d, s), jnp.float32),
        compiler_params=pltpu.CompilerParams(use_tc_tiling_on_sc=True,
                                             needs_layout_passes=False),
        scratch_types=[
            pltpu.VMEM((tpw,), jnp.int32),
            pltpu.VMEM((hd, k), jnp.float32),
            pltpu.VMEM((hd, tpw), jnp.float32),
        ],
    )
    def gk(wt_hbm, idx_hbm, out_hbm, idx_v, wt_v, out_v):
        wid = lax.axis_index("s") * info.num_cores + lax.axis_index("c")
        h = wid % 2
        rng = wid // 2
        tpb = tpw // s
        bb0 = rng * tpb
        pltpu.sync_copy(wt_hbm.at[pl.ds(h * hd, hd)], wt_v)
        pltpu.sync_copy(idx_hbm.at[pl.ds(rng * tpw, tpw)], idx_v)
        rows = [jnp.full((nl,), dd, jnp.int32) for dd in range(hd)]

        @plsc.parallel_loop(0, tpw // nl, unroll=2)
        def _gather_group(gg):
            toks = idx_v[pl.ds(gg * nl, nl)]
            for dd in range(hd):
                out_v[dd, pl.ds(gg * nl, nl)] = plsc.load_gather(
                    wt_v, [rows[dd], toks])

        for b2 in range(tpb):
            pltpu.sync_copy(out_v.at[pl.ds(0, hd), pl.ds(b2 * s, s)],
                            out_hbm.at[bb0 + b2, pl.ds(h * hd, hd)])

    return gk(wt, idx)


def kernel(z, W):
    b, s, d = z.shape
    k = W.shape[0]
    zt = jnp.transpose(z, (0, 2, 1))
    wt = jnp.transpose(W, (1, 0))

    nh = b // _SPLIT
    scale = 1.25 / (b * s * d)
    idx_parts, loss_parts, qt_parts = [], [], []
    for p in range(_SPLIT):
        idx_p, lacc_p = _tc_half(zt, wt, p * nh, nh, scale)
        qt_parts.append(_sc_gather_t(wt, idx_p, nh, s))
        idx_parts.append(idx_p)
        loss_parts.append(lacc_p[0, 0])

    qt = jnp.concatenate(qt_parts, axis=0)
    idx = jnp.concatenate(idx_parts).reshape(b, s)
    loss = loss_parts[0]
    for lp in loss_parts[1:]:
        loss = loss + lp
    return jnp.transpose(qt, (0, 2, 1)), loss, idx


# speedup vs baseline: 1.5347x; 1.0230x over previous
import functools

import jax
import jax.numpy as jnp
from jax import lax
from jax.experimental import pallas as pl
from jax.experimental.pallas import tpu as pltpu
from jax.experimental.pallas import tpu_sc as plsc

_CH = 128
_SPLIT = 1


def _vq_tc_body(scale, d, rb, zt_ref, wt_ref, idx_ref, loss_ref):
    pid = pl.program_id(0)
    wt = wt_ref[...]
    k = wt.shape[1]
    s = zt_ref.shape[-1]
    wsq = jnp.sum(wt * wt, axis=0)
    wsq_col = wsq.reshape(k, 1)
    row_f = lax.broadcasted_iota(jnp.int32, (_CH, s), 0).astype(jnp.float32)
    msum = jnp.zeros((), jnp.float32)
    for bi in range(rb):
        zt = zt_ref[bi]
        zsq = jnp.sum(zt * zt, axis=0)
        mm = lax.dot_general(wt, zt, (((0,), (0,)), ((), ())),
                             preferred_element_type=jnp.float32)
        dist = (zsq[None, :] + wsq_col) - 2.0 * mm
        val = dist[0:_CH, :]
        chk = jnp.zeros((_CH, s), jnp.float32)
        for j in range(1, k // _CH):
            dj = dist[j * _CH:(j + 1) * _CH, :]
            cond = dj < val
            val = jnp.minimum(val, dj)
            chk = jnp.where(cond, jnp.float32(j), chk)
        m = jnp.min(val, axis=0)
        g = chk * jnp.float32(_CH) + row_f
        idxf = jnp.min(jnp.where(val == m[None, :], g, jnp.float32(k)),
                       axis=0)
        idx_ref[pl.ds(bi * s, s)] = idxf.astype(jnp.int32)
        msum = msum + jnp.sum(m)

    @pl.when(pid == 0)
    def _init():
        loss_ref[...] = jnp.zeros((1, 1), jnp.float32)

    loss_ref[...] += msum.reshape(1, 1)

    @pl.when(pid == pl.num_programs(0) - 1)
    def _finish():
        loss_ref[...] *= scale


def _tc_half(zt, wt, off, nb, scale, rb=4):
    d, k = wt.shape
    s = zt.shape[-1]
    return pl.pallas_call(
        functools.partial(_vq_tc_body, scale, d, rb),
        grid=(nb // rb,),
        in_specs=[
            pl.BlockSpec((rb, d, s), lambda i, off=off: (i + off, 0, 0)),
            pl.BlockSpec((d, k), lambda i: (0, 0)),
        ],
        out_specs=[
            pl.BlockSpec((rb * s,), lambda i: (i,)),
            pl.BlockSpec((1, 1), lambda i: (0, 0)),
        ],
        out_shape=[
            jax.ShapeDtypeStruct((nb * s,), jnp.int32),
            jax.ShapeDtypeStruct((1, 1), jnp.float32),
        ],
    )(zt, wt)


def _sc_gather_t(wt, idx, nb, s):
    d, k = wt.shape
    hd = d // 2
    nt = nb * s
    info = plsc.get_sparse_core_info()
    nl = info.num_lanes
    nw = info.num_cores * info.num_subcores
    nrng = nw // 2
    tpw = nt // nrng
    mesh = plsc.VectorSubcoreMesh(core_axis_name="c", subcore_axis_name="s")

    @functools.partial(
        pl.kernel, mesh=mesh,
        out_type=jax.ShapeDtypeStruct((nb, d, s), jnp.float32),
        compiler_params=pltpu.CompilerParams(use_tc_tiling_on_sc=True,
                                             needs_layout_passes=False),
        scratch_types=[
            pltpu.VMEM((tpw,), jnp.int32),
            pltpu.VMEM((hd, k), jnp.float32),
            pltpu.VMEM((hd, tpw), jnp.float32),
        ],
    )
    def gk(wt_hbm, idx_hbm, out_hbm, idx_v, wt_v, out_v):
        wid = lax.axis_index("s") * info.num_cores + lax.axis_index("c")
        h = wid % 2
        rng = wid // 2
        tpb = tpw // s
        bb0 = rng * tpb
        pltpu.sync_copy(wt_hbm.at[pl.ds(h * hd, hd)], wt_v)
        pltpu.sync_copy(idx_hbm.at[pl.ds(rng * tpw, tpw)], idx_v)
        rows = [jnp.full((nl,), dd, jnp.int32) for dd in range(hd)]

        @plsc.parallel_loop(0, tpw // nl, unroll=2)
        def _gather_group(gg):
            toks = idx_v[pl.ds(gg * nl, nl)]
            for dd in range(hd):
                out_v[dd, pl.ds(gg * nl, nl)] = plsc.load_gather(
                    wt_v, [rows[dd], toks])

        for b2 in range(tpb):
            pltpu.sync_copy(out_v.at[pl.ds(0, hd), pl.ds(b2 * s, s)],
                            out_hbm.at[bb0 + b2, pl.ds(h * hd, hd)])

    return gk(wt, idx)


def kernel(z, W):
    b, s, d = z.shape
    k = W.shape[0]
    zt = jnp.transpose(z, (0, 2, 1))
    wt = jnp.transpose(W, (1, 0))

    nh = b // _SPLIT
    scale = 1.25 / (b * s * d)
    idx_parts, loss_parts, qt_parts = [], [], []
    for p in range(_SPLIT):
        idx_p, lacc_p = _tc_half(zt, wt, p * nh, nh, scale)
        qt_parts.append(_sc_gather_t(wt, idx_p, nh, s))
        idx_parts.append(idx_p)
        loss_parts.append(lacc_p[0, 0])

    qt = jnp.concatenate(qt_parts, axis=0)
    idx = jnp.concatenate(idx_parts).reshape(b, s)
    loss = loss_parts[0]
    for lp in loss_parts[1:]:
        loss = loss + lp
    return jnp.transpose(qt, (0, 2, 1)), loss, idx
